# Initial kernel scaffold; baseline (speedup 1.0000x reference)
import jax
import jax.numpy as jnp
from jax.experimental import pallas as pl


def kernel(x, edge_index, batch, W1, b1, g1, be1, W2, b2, g2, be2, W3, b3, g3, be3, W4, b4, g4, be4, W5, b5, g5, be5, W6, b6, g6, be6, W7, b7, g7, be7, fW1, fb1, fW2, fb2):
    raise NotImplementedError("write your pallas kernel here")


# speedup vs baseline: 4.9037x; 4.9037x over previous
import functools

import jax
import jax.numpy as jnp
from jax import lax
from jax.experimental import pallas as pl
from jax.experimental.pallas import tpu as pltpu
from jax.experimental.pallas import tpu_sc as plsc

N = 10000
E = 320000
G = 64
NCORES = 2
NSUB = 16
NW = NCORES * NSUB
CE = 128
NCHP = 80
EP = NW * NCHP * CE
NA = N + 8
RPT = 624
TAIL0 = NSUB * RPT
BN = 1.0 / (1.0 + 1e-5) ** 0.5
R = 1000
GRID = N // R
FP = 128


def _sc_mesh():
    return plsc.VectorSubcoreMesh(
        core_axis_name="c", subcore_axis_name="s",
        num_cores=NCORES, num_subcores=NSUB)


def _make_agg():
    @functools.partial(
        pl.kernel,
        out_type=jax.ShapeDtypeStruct((NCORES, N, FP), jnp.float32),
        mesh=_sc_mesh(),
        scratch_types=[
            pltpu.VMEM((NCHP, CE), jnp.int32),
            pltpu.VMEM((NCHP, CE), jnp.int32),
            pltpu.VMEM((CE, FP), jnp.float32),
            pltpu.MemorySpace.VMEM_SHARED((NA, FP), jnp.float32),
            pltpu.SemaphoreType.DMA,
        ],
    )
    def agg_kernel(table_h, sidx_h, didx_h, zeros_h, out_h,
                   sidx_v, didx_v, rows_v, acc, sem):
        c = lax.axis_index("c")
        sid = lax.axis_index("s")
        wid = c * NSUB + sid
        pltpu.sync_copy(zeros_h.at[pl.ds(sid * RPT, RPT)],
                        acc.at[pl.ds(sid * RPT, RPT)])

        @pl.when(sid == NSUB - 1)
        def _():
            pltpu.sync_copy(zeros_h.at[pl.ds(TAIL0, NA - TAIL0)],
                            acc.at[pl.ds(TAIL0, NA - TAIL0)])

        pltpu.sync_copy(sidx_h.at[pl.ds(wid * NCHP, NCHP)], sidx_v)
        pltpu.sync_copy(didx_h.at[pl.ds(wid * NCHP, NCHP)], didx_v)
        plsc.subcore_barrier()

        def step(j, carry):
            pltpu.async_copy(table_h.at[sidx_v.at[j]], rows_v, sem).wait()
            pltpu.sync_copy(rows_v, acc.at[didx_v.at[j]], add=True)
            return carry

        lax.fori_loop(0, NCHP, step, 0)
        plsc.subcore_barrier()
        pltpu.sync_copy(acc.at[pl.ds(sid * RPT, RPT)],
                        out_h.at[c, pl.ds(sid * RPT, RPT)])

        @pl.when(sid == NSUB - 1)
        def _():
            pltpu.sync_copy(acc.at[pl.ds(TAIL0, N - TAIL0)],
                            out_h.at[c, pl.ds(TAIL0, N - TAIL0)])

    return agg_kernel


def _row_spec(shape):
    if len(shape) == 3:
        return pl.BlockSpec((shape[0], R, shape[2]), lambda i: (0, i, 0))
    return pl.BlockSpec((R, shape[1]), lambda i: (i, 0))


def _full_spec(shape):
    nd = len(shape)
    return pl.BlockSpec(shape, lambda i, _nd=nd: (0,) * _nd)


def _tc_call(body, ins, row_in, out_shapes):
    in_specs = [
        _row_spec(a.shape) if rb else _full_spec(a.shape)
        for a, rb in zip(ins, row_in)
    ]
    out_specs = [_row_spec(s.shape) for s in out_shapes]
    single = len(out_shapes) == 1
    return pl.pallas_call(
        body,
        grid=(GRID,),
        in_specs=in_specs,
        out_specs=out_specs[0] if single else out_specs,
        out_shape=out_shapes[0] if single else out_shapes,
    )(*ins)


def _sigbn(z, g_ref, be_ref):
    return jax.nn.sigmoid(z) * (g_ref[...] * BN) + be_ref[...]


def _dot(a, b):
    return jnp.dot(a, b, preferred_element_type=jnp.float32)


def _pad128(x):
    r, f = x.shape
    return jnp.concatenate([x, jnp.zeros((r, FP - f), jnp.float32)], axis=1)


def _k_dinv(u_ref, x_ref, dinv_ref, xt_ref):
    deg = u_ref[0, :, :1] + u_ref[1, :, :1] + 1.0
    dinv = lax.rsqrt(deg)
    dinv_ref[...] = dinv
    xt_ref[...] = _pad128(dinv * x_ref[...])


def _k_l1(u_ref, xt_ref, dinv_ref, W_ref, b_ref, g_ref, be_ref, ht_ref):
    dv = dinv_ref[...]
    a = dv * (u_ref[0, :, :2] + u_ref[1, :, :2] + xt_ref[:, :2])
    h = _sigbn(_dot(a, W_ref[...]) + b_ref[...], g_ref, be_ref)
    ht_ref[...] = _pad128(dv * h)


def _k_l2(u_ref, ht1_ref, dinv_ref, W_ref, b_ref, g_ref, be_ref, ht2_ref):
    dv = dinv_ref[...]
    a2 = dv * (u_ref[0, :, :64] + u_ref[1, :, :64] + ht1_ref[:, :64])
    h = _sigbn(_dot(a2, W_ref[...]) + b_ref[...], g_ref, be_ref)
    ht2_ref[...] = _pad128(dv * h)


def _k_l34(u2_ref, ht1_ref, u3_ref, ht2_ref, dinv_ref, W3_ref, b3_ref,
           g3_ref, be3_ref, W4_ref, t4a_ref, t4b_ref):
    dv = dinv_ref[...]
    a2 = dv * (u2_ref[0, :, :64] + u2_ref[1, :, :64] + ht1_ref[:, :64])
    s3 = dv * (u3_ref[0, :, :64] + u3_ref[1, :, :64] + ht2_ref[:, :64])
    z3 = _dot(a2, W3_ref[:64]) + _dot(s3, W3_ref[64:]) + b3_ref[...]
    h3 = _sigbn(z3, g3_ref, be3_ref)
    t4 = dv * _dot(h3, W4_ref[...])
    t4a_ref[...] = t4[:, :128]
    t4b_ref[...] = t4[:, 128:]


def _k_l45(u4a_ref, u4b_ref, t4a_ref, t4b_ref, dinv_ref, b4_ref, g4_ref,
           be4_ref, W5_ref, t5_ref):
    dv = dinv_ref[...]
    z4a = dv * (u4a_ref[0] + u4a_ref[1] + t4a_ref[...])
    z4b = dv * (u4b_ref[0] + u4b_ref[1] + t4b_ref[...])
    z4 = jnp.concatenate([z4a, z4b], axis=1) + b4_ref[...]
    h4 = _sigbn(z4, g4_ref, be4_ref)
    t5_ref[...] = _pad128(dv * _dot(h4, W5_ref[...]))


def _k_mid(F_in):
    def body(u_ref, t_ref, dinv_ref, b_ref, g_ref, be_ref, Wn_ref, tn_ref):
        dv = dinv_ref[...]
        z = dv * (u_ref[0, :, :F_in] + u_ref[1, :, :F_in]
                  + t_ref[:, :F_in]) + b_ref[...]
        h = _sigbn(z, g_ref, be_ref)
        tn_ref[...] = _pad128(dv * _dot(h, Wn_ref[...]))
    return body


def _k_head(u7_ref, t7_ref, dinv_ref, b7_ref, g7_ref, be7_ref, batch_ref,
            fW1_ref, fb1_ref, fW2_ref, fb2_ref, out_ref):
    dv = dinv_ref[...]
    z7 = dv * (u7_ref[0, :, :16] + u7_ref[1, :, :16]
               + t7_ref[:, :16]) + b7_ref[...]
    h7 = _sigbn(z7, g7_ref, be7_ref)
    bi = batch_ref[...]
    ohT = (lax.broadcasted_iota(jnp.int32, (G, N), 0) == bi)
    ohT = ohT.astype(jnp.float32)
    pooled = _dot(ohT, h7)
    cnt = _dot(ohT, jnp.ones((N, 1), jnp.float32))
    pooled = pooled / jnp.maximum(cnt, 1.0)
    hf = jax.nn.sigmoid(_dot(pooled, fW1_ref[...]) + fb1_ref[...])
    out_ref[...] = jax.nn.sigmoid(_dot(hf, fW2_ref[...]) + fb2_ref[...])


def kernel(x, edge_index, batch, W1, b1, g1, be1, W2, b2, g2, be2, W3, b3,
           g3, be3, W4, b4, g4, be4, W5, b5, g5, be5, W6, b6, g6, be6, W7,
           b7, g7, be7, fW1, fb1, fW2, fb2):
    f32 = jnp.float32
    pad = EP - E
    sidx = jnp.concatenate([edge_index[0], jnp.zeros((pad,), jnp.int32)])
    didx = jnp.concatenate(
        [edge_index[1], jnp.full((pad,), N, jnp.int32)])
    sidx = sidx.reshape(NW * NCHP, CE)
    didx = didx.reshape(NW * NCHP, CE)
    zeros_acc = jnp.zeros((NA, FP), f32)

    agg_k = _make_agg()

    def agg(table):
        return agg_k(table, sidx, didx, zeros_acc)

    u0 = agg(jnp.ones((N, FP), f32))
    dinv, xt = _tc_call(
        _k_dinv, [u0, x], [True, True],
        [jax.ShapeDtypeStruct((N, 1), f32),
         jax.ShapeDtypeStruct((N, FP), f32)])

    rb = lambda k: [True] * k
    b1r, g1r, be1r = b1.reshape(1, -1), g1.reshape(1, -1), be1.reshape(1, -1)
    b2r, g2r, be2r = b2.reshape(1, -1), g2.reshape(1, -1), be2.reshape(1, -1)
    b3r, g3r, be3r = b3.reshape(1, -1), g3.reshape(1, -1), be3.reshape(1, -1)
    b4r, g4r, be4r = b4.reshape(1, -1), g4.reshape(1, -1), be4.reshape(1, -1)
    b5r, g5r, be5r = b5.reshape(1, -1), g5.reshape(1, -1), be5.reshape(1, -1)
    b6r, g6r, be6r = b6.reshape(1, -1), g6.reshape(1, -1), be6.reshape(1, -1)
    b7r, g7r, be7r = b7.reshape(1, -1), g7.reshape(1, -1), be7.reshape(1, -1)

    u1 = agg(xt)
    ht1 = _tc_call(
        _k_l1, [u1, xt, dinv, W1, b1r, g1r, be1r],
        rb(3) + [False] * 4, [jax.ShapeDtypeStruct((N, FP), f32)])
    u2 = agg(ht1)
    ht2 = _tc_call(
        _k_l2, [u2, ht1, dinv, W2, b2r, g2r, be2r],
        rb(3) + [False] * 4, [jax.ShapeDtypeStruct((N, FP), f32)])
    u3 = agg(ht2)
    t4a, t4b = _tc_call(
        _k_l34, [u2, ht1, u3, ht2, dinv, W3, b3r, g3r, be3r, W4],
        rb(5) + [False] * 5,
        [jax.ShapeDtypeStruct((N, FP), f32),
         jax.ShapeDtypeStruct((N, FP), f32)])
    u4a = agg(t4a)
    u4b = agg(t4b)
    t5 = _tc_call(
        _k_l45, [u4a, u4b, t4a, t4b, dinv, b4r, g4r, be4r, W5],
        rb(5) + [False] * 4, [jax.ShapeDtypeStruct((N, FP), f32)])
    u5 = agg(t5)
    t6 = _tc_call(
        _k_mid(64), [u5, t5, dinv, b5r, g5r, be5r, W6],
        rb(3) + [False] * 4, [jax.ShapeDtypeStruct((N, FP), f32)])
    u6 = agg(t6)
    t7 = _tc_call(
        _k_mid(32), [u6, t6, dinv, b6r, g6r, be6r, W7],
        rb(3) + [False] * 4, [jax.ShapeDtypeStruct((N, FP), f32)])
    u7 = agg(t7)
    out = pl.pallas_call(
        _k_head,
        out_shape=jax.ShapeDtypeStruct((G, 2), f32),
    )(u7, t7, dinv, b7r, g7r, be7r, batch.reshape(1, N),
      fW1, fb1.reshape(1, -1), fW2, fb2.reshape(1, -1))
    return out


# speedup vs baseline: 5.3077x; 1.0824x over previous
import functools

import jax
import jax.numpy as jnp
from jax import lax
from jax.experimental import pallas as pl
from jax.experimental.pallas import tpu as pltpu
from jax.experimental.pallas import tpu_sc as plsc

N = 10000
E = 320000
G = 64
NCORES = 2
NSUB = 16
NW = NCORES * NSUB
CE = 128
NCHP = 80
EP = NW * NCHP * CE
NA = N + 8
RPT = 624
TAIL0 = NSUB * RPT
BN = 1.0 / (1.0 + 1e-5) ** 0.5
R = 1000
GRID = N // R
FP = 128


def _sc_mesh():
    return plsc.VectorSubcoreMesh(
        core_axis_name="c", subcore_axis_name="s",
        num_cores=NCORES, num_subcores=NSUB)


def _make_agg():
    @functools.partial(
        pl.kernel,
        out_type=jax.ShapeDtypeStruct((NCORES, N, FP), jnp.float32),
        mesh=_sc_mesh(),
        scratch_types=[
            pltpu.VMEM((NCHP // 2, CE), jnp.int32),
            pltpu.VMEM((NCHP // 2, CE), jnp.int32),
            pltpu.VMEM((CE, FP), jnp.float32),
            pltpu.VMEM((CE, FP), jnp.float32),
            pltpu.MemorySpace.VMEM_SHARED((NA, FP), jnp.float32),
            pltpu.SemaphoreType.DMA,
            pltpu.SemaphoreType.DMA,
        ],
    )
    def agg_kernel(table_h, sidx_h, didx_h, zeros_h, out_h,
                   sidx_v, didx_v, rows_a, rows_b, acc, sem_a, sem_b):
        c = lax.axis_index("c")
        sid = lax.axis_index("s")
        wid = c * NSUB + sid
        pltpu.sync_copy(zeros_h.at[pl.ds(sid * RPT, RPT)],
                        acc.at[pl.ds(sid * RPT, RPT)])

        @pl.when(sid == NSUB - 1)
        def _():
            pltpu.sync_copy(zeros_h.at[pl.ds(TAIL0, NA - TAIL0)],
                            acc.at[pl.ds(TAIL0, NA - TAIL0)])

        plsc.subcore_barrier()

        HALF = NCHP // 2
        for p in range(2):
            pltpu.sync_copy(sidx_h.at[pl.ds(wid * NCHP + p * HALF, HALF)],
                            sidx_v)
            pltpu.sync_copy(didx_h.at[pl.ds(wid * NCHP + p * HALF, HALF)],
                            didx_v)
            pltpu.async_copy(table_h.at[sidx_v.at[0]], rows_a, sem_a)

            def step2(i, carry):
                j = 2 * i
                pltpu.make_async_copy(table_h.at[sidx_v.at[j]],
                                      rows_a, sem_a).wait()
                pltpu.async_copy(table_h.at[sidx_v.at[j + 1]], rows_b, sem_b)
                pltpu.sync_copy(rows_a, acc.at[didx_v.at[j]], add=True)
                pltpu.make_async_copy(table_h.at[sidx_v.at[j + 1]],
                                      rows_b, sem_b).wait()

                @pl.when(i < HALF // 2 - 1)
                def _():
                    pltpu.async_copy(table_h.at[sidx_v.at[j + 2]],
                                     rows_a, sem_a)

                pltpu.sync_copy(rows_b, acc.at[didx_v.at[j + 1]], add=True)
                return carry

            lax.fori_loop(0, HALF // 2, step2, 0)
        plsc.subcore_barrier()
        pltpu.sync_copy(acc.at[pl.ds(sid * RPT, RPT)],
                        out_h.at[c, pl.ds(sid * RPT, RPT)])

        @pl.when(sid == NSUB - 1)
        def _():
            pltpu.sync_copy(acc.at[pl.ds(TAIL0, N - TAIL0)],
                            out_h.at[c, pl.ds(TAIL0, N - TAIL0)])

    return agg_kernel


def _row_spec(shape):
    if len(shape) == 3:
        return pl.BlockSpec((shape[0], R, shape[2]), lambda i: (0, i, 0))
    return pl.BlockSpec((R, shape[1]), lambda i: (i, 0))


def _full_spec(shape):
    nd = len(shape)
    return pl.BlockSpec(shape, lambda i, _nd=nd: (0,) * _nd)


def _tc_call(body, ins, row_in, out_shapes):
    in_specs = [
        _row_spec(a.shape) if rb else _full_spec(a.shape)
        for a, rb in zip(ins, row_in)
    ]
    out_specs = [_row_spec(s.shape) for s in out_shapes]
    single = len(out_shapes) == 1
    return pl.pallas_call(
        body,
        grid=(GRID,),
        in_specs=in_specs,
        out_specs=out_specs[0] if single else out_specs,
        out_shape=out_shapes[0] if single else out_shapes,
    )(*ins)


def _sigbn(z, g_ref, be_ref):
    return jax.nn.sigmoid(z) * (g_ref[...] * BN) + be_ref[...]


def _dot(a, b):
    return jnp.dot(a, b, preferred_element_type=jnp.float32)


def _pad128(x):
    r, f = x.shape
    return jnp.concatenate([x, jnp.zeros((r, FP - f), jnp.float32)], axis=1)


def _k_dinv(u_ref, x_ref, dinv_ref, xt_ref):
    deg = u_ref[0, :, :1] + u_ref[1, :, :1] + 1.0
    dinv = lax.rsqrt(deg)
    dinv_ref[...] = dinv
    xt_ref[...] = _pad128(dinv * x_ref[...])


def _k_l1(u_ref, xt_ref, dinv_ref, W_ref, b_ref, g_ref, be_ref, ht_ref):
    dv = dinv_ref[...]
    a = dv * (u_ref[0, :, :2] + u_ref[1, :, :2] + xt_ref[:, :2])
    h = _sigbn(_dot(a, W_ref[...]) + b_ref[...], g_ref, be_ref)
    ht_ref[...] = _pad128(dv * h)


def _k_l2(u_ref, ht1_ref, dinv_ref, W_ref, b_ref, g_ref, be_ref, ht2_ref):
    dv = dinv_ref[...]
    a2 = dv * (u_ref[0, :, :64] + u_ref[1, :, :64] + ht1_ref[:, :64])
    h = _sigbn(_dot(a2, W_ref[...]) + b_ref[...], g_ref, be_ref)
    ht2_ref[...] = _pad128(dv * h)


def _k_l34(u2_ref, ht1_ref, u3_ref, ht2_ref, dinv_ref, W3_ref, b3_ref,
           g3_ref, be3_ref, W4_ref, t4a_ref, t4b_ref):
    dv = dinv_ref[...]
    a2 = dv * (u2_ref[0, :, :64] + u2_ref[1, :, :64] + ht1_ref[:, :64])
    s3 = dv * (u3_ref[0, :, :64] + u3_ref[1, :, :64] + ht2_ref[:, :64])
    z3 = _dot(a2, W3_ref[:64]) + _dot(s3, W3_ref[64:]) + b3_ref[...]
    h3 = _sigbn(z3, g3_ref, be3_ref)
    t4 = dv * _dot(h3, W4_ref[...])
    t4a_ref[...] = t4[:, :128]
    t4b_ref[...] = t4[:, 128:]


def _k_l45(u4a_ref, u4b_ref, t4a_ref, t4b_ref, dinv_ref, b4_ref, g4_ref,
           be4_ref, W5_ref, t5_ref):
    dv = dinv_ref[...]
    z4a = dv * (u4a_ref[0] + u4a_ref[1] + t4a_ref[...])
    z4b = dv * (u4b_ref[0] + u4b_ref[1] + t4b_ref[...])
    z4 = jnp.concatenate([z4a, z4b], axis=1) + b4_ref[...]
    h4 = _sigbn(z4, g4_ref, be4_ref)
    t5_ref[...] = _pad128(dv * _dot(h4, W5_ref[...]))


def _k_mid(F_in):
    def body(u_ref, t_ref, dinv_ref, b_ref, g_ref, be_ref, Wn_ref, tn_ref):
        dv = dinv_ref[...]
        z = dv * (u_ref[0, :, :F_in] + u_ref[1, :, :F_in]
                  + t_ref[:, :F_in]) + b_ref[...]
        h = _sigbn(z, g_ref, be_ref)
        tn_ref[...] = _pad128(dv * _dot(h, Wn_ref[...]))
    return body


def _k_head(u7_ref, t7_ref, dinv_ref, b7_ref, g7_ref, be7_ref, batch_ref,
            fW1_ref, fb1_ref, fW2_ref, fb2_ref, out_ref):
    dv = dinv_ref[...]
    z7 = dv * (u7_ref[0, :, :16] + u7_ref[1, :, :16]
               + t7_ref[:, :16]) + b7_ref[...]
    h7 = _sigbn(z7, g7_ref, be7_ref)
    bi = batch_ref[...]
    ohT = (lax.broadcasted_iota(jnp.int32, (G, N), 0) == bi)
    ohT = ohT.astype(jnp.float32)
    pooled = _dot(ohT, h7)
    cnt = _dot(ohT, jnp.ones((N, 1), jnp.float32))
    pooled = pooled / jnp.maximum(cnt, 1.0)
    hf = jax.nn.sigmoid(_dot(pooled, fW1_ref[...]) + fb1_ref[...])
    out_ref[...] = jax.nn.sigmoid(_dot(hf, fW2_ref[...]) + fb2_ref[...])


def kernel(x, edge_index, batch, W1, b1, g1, be1, W2, b2, g2, be2, W3, b3,
           g3, be3, W4, b4, g4, be4, W5, b5, g5, be5, W6, b6, g6, be6, W7,
           b7, g7, be7, fW1, fb1, fW2, fb2):
    f32 = jnp.float32
    pad = EP - E
    sidx = jnp.concatenate([edge_index[0], jnp.zeros((pad,), jnp.int32)])
    didx = jnp.concatenate(
        [edge_index[1], jnp.full((pad,), N, jnp.int32)])
    sidx = sidx.reshape(NW * NCHP, CE)
    didx = didx.reshape(NW * NCHP, CE)
    zeros_acc = jnp.zeros((NA, FP), f32)

    agg_k = _make_agg()

    def agg(table):
        return agg_k(table, sidx, didx, zeros_acc)

    u0 = agg(jnp.ones((N, FP), f32))
    dinv, xt = _tc_call(
        _k_dinv, [u0, x], [True, True],
        [jax.ShapeDtypeStruct((N, 1), f32),
         jax.ShapeDtypeStruct((N, FP), f32)])

    rb = lambda k: [True] * k
    b1r, g1r, be1r = b1.reshape(1, -1), g1.reshape(1, -1), be1.reshape(1, -1)
    b2r, g2r, be2r = b2.reshape(1, -1), g2.reshape(1, -1), be2.reshape(1, -1)
    b3r, g3r, be3r = b3.reshape(1, -1), g3.reshape(1, -1), be3.reshape(1, -1)
    b4r, g4r, be4r = b4.reshape(1, -1), g4.reshape(1, -1), be4.reshape(1, -1)
    b5r, g5r, be5r = b5.reshape(1, -1), g5.reshape(1, -1), be5.reshape(1, -1)
    b6r, g6r, be6r = b6.reshape(1, -1), g6.reshape(1, -1), be6.reshape(1, -1)
    b7r, g7r, be7r = b7.reshape(1, -1), g7.reshape(1, -1), be7.reshape(1, -1)

    u1 = agg(xt)
    ht1 = _tc_call(
        _k_l1, [u1, xt, dinv, W1, b1r, g1r, be1r],
        rb(3) + [False] * 4, [jax.ShapeDtypeStruct((N, FP), f32)])
    u2 = agg(ht1)
    ht2 = _tc_call(
        _k_l2, [u2, ht1, dinv, W2, b2r, g2r, be2r],
        rb(3) + [False] * 4, [jax.ShapeDtypeStruct((N, FP), f32)])
    u3 = agg(ht2)
    t4a, t4b = _tc_call(
        _k_l34, [u2, ht1, u3, ht2, dinv, W3, b3r, g3r, be3r, W4],
        rb(5) + [False] * 5,
        [jax.ShapeDtypeStruct((N, FP), f32),
         jax.ShapeDtypeStruct((N, FP), f32)])
    u4a = agg(t4a)
    u4b = agg(t4b)
    t5 = _tc_call(
        _k_l45, [u4a, u4b, t4a, t4b, dinv, b4r, g4r, be4r, W5],
        rb(5) + [False] * 4, [jax.ShapeDtypeStruct((N, FP), f32)])
    u5 = agg(t5)
    t6 = _tc_call(
        _k_mid(64), [u5, t5, dinv, b5r, g5r, be5r, W6],
        rb(3) + [False] * 4, [jax.ShapeDtypeStruct((N, FP), f32)])
    u6 = agg(t6)
    t7 = _tc_call(
        _k_mid(32), [u6, t6, dinv, b6r, g6r, be6r, W7],
        rb(3) + [False] * 4, [jax.ShapeDtypeStruct((N, FP), f32)])
    u7 = agg(t7)
    out = pl.pallas_call(
        _k_head,
        out_shape=jax.ShapeDtypeStruct((G, 2), f32),
    )(u7, t7, dinv, b7r, g7r, be7r, batch.reshape(1, N),
      fW1, fb1.reshape(1, -1), fW2, fb2.reshape(1, -1))
    return out
